# baseline (speedup 1.0000x reference)
import jax
import jax.numpy as jnp
from jax.experimental import pallas as pl
from jax.experimental.pallas import tpu as pltpu
from jax.experimental.pallas import tpu_sc as plsc

B = 1
T = 2048
D = 768
H = 12
KV = 4
HD = 64
E = 8
TOPK = 2
FF = 2048
EPS = 1e-5
THETA = 1000000.0

TBLK = 256
RB = 256
NP = T * TOPK
PAD_N = NP + E * RB
NBLK = PAD_N // RB
NEG = -1e9

def _vmesh():
    return plsc.VectorSubcoreMesh(core_axis_name="core",
                                  subcore_axis_name="subcore")


def _rms(x, w):
    return x * jax.lax.rsqrt(jnp.mean(x * x, axis=-1, keepdims=True) + EPS) * w


def _qkv_kernel(x_ref, pos_ref, ln1_ref, wq_ref, wk_ref, wv_ref,
                q_ref, k_ref, v_ref):
    x = x_ref[...]
    h = _rms(x, ln1_ref[...]).astype(jnp.bfloat16)
    pos = pos_ref[...].astype(jnp.float32)
    inv_freq = 1.0 / (THETA ** (
        jax.lax.broadcasted_iota(jnp.int32, (1, HD // 2), 1).astype(jnp.float32)
        * (2.0 / HD)))
    freqs = pos * inv_freq
    emb = jnp.concatenate([freqs, freqs], axis=-1)
    cos = jnp.cos(emb)[:, None, :]
    sin = jnp.sin(emb)[:, None, :]

    def rope(y, nheads, scale):
        y3 = y.reshape(TBLK, nheads, HD)
        yr = jnp.concatenate([-y3[..., HD // 2:], y3[..., :HD // 2]], axis=-1)
        return ((y3 * cos + yr * sin) * scale).reshape(TBLK, nheads * HD)

    q = jnp.dot(h, wq_ref[...].astype(jnp.bfloat16),
                preferred_element_type=jnp.float32)
    k = jnp.dot(h, wk_ref[...].astype(jnp.bfloat16),
                preferred_element_type=jnp.float32)
    v = jnp.dot(h, wv_ref[...].astype(jnp.bfloat16),
                preferred_element_type=jnp.float32)
    q_ref[...] = rope(q, H, 1.0 / (HD ** 0.5)).astype(jnp.bfloat16)
    k_ref[...] = rope(k, KV, 1.0).astype(jnp.bfloat16)
    v_ref[...] = v.astype(jnp.bfloat16)


def _attn_kernel(q_ref, k_ref, v_ref, o_ref):
    qb = pl.program_id(0)
    q = q_ref[0]
    k = k_ref[0]
    s = jax.lax.dot_general(q, k, (((1,), (1,)), ((), ())),
                            preferred_element_type=jnp.float32)
    row = qb * TBLK + jax.lax.broadcasted_iota(jnp.int32, (TBLK, T), 0)
    col = jax.lax.broadcasted_iota(jnp.int32, (TBLK, T), 1)
    s = jnp.where(col <= row, s, NEG)
    m = jnp.max(s, axis=-1, keepdims=True)
    p = jnp.exp(s - m)
    p = p / jnp.sum(p, axis=-1, keepdims=True)
    o_ref[0] = jnp.dot(p.astype(jnp.bfloat16), v_ref[0],
                       preferred_element_type=jnp.float32).astype(jnp.bfloat16)


def _out_kernel(a_ref, wo_ref, x_ref, ln2_ref, x1_ref, h2_ref):
    ao = jnp.dot(a_ref[...], wo_ref[...].astype(jnp.bfloat16),
                 preferred_element_type=jnp.float32)
    x1 = x_ref[...] + ao
    h2 = _rms(x1, ln2_ref[...])
    x1_ref[...] = x1
    h2_ref[...] = h2


def _router_kernel(h2_ref, wr_ref, ti_ref, tw_ref):
    logits = jnp.dot(h2_ref[...], wr_ref[...],
                     preferred_element_type=jnp.float32)
    m = jnp.max(logits, axis=-1, keepdims=True)
    p = jnp.exp(logits - m)
    p = p / jnp.sum(p, axis=-1, keepdims=True)
    lane = jax.lax.broadcasted_iota(jnp.int32, (T, E), 1)
    i1 = jnp.argmax(p, axis=-1, keepdims=True)
    m1 = jnp.max(p, axis=-1, keepdims=True)
    p2 = jnp.where(lane == i1, -1.0, p)
    i2 = jnp.argmax(p2, axis=-1, keepdims=True)
    m2 = jnp.max(p2, axis=-1, keepdims=True)
    denom = m1 + m2
    ti_ref[...] = jnp.concatenate([i1, i2], axis=1).astype(jnp.int32)
    tw_ref[...] = jnp.concatenate([m1 / denom, m2 / denom], axis=1)


def _sc_gather(src, idx, out_dtype, win=128):
    n, c = src.shape
    m = idx.shape[0]
    sub = c // 128
    src2 = src.reshape(n * sub, 128)
    idxe = (idx[:, None] * sub
            + jnp.arange(sub, dtype=jnp.int32)[None, :]).reshape(1, m * sub)

    @pl.kernel(out_type=jax.ShapeDtypeStruct((m * sub, 128), out_dtype),
               mesh=_vmesh())
    def k(x_hbm, i_hbm, o_hbm):
        def body(i_vmem, o_vmem):
            pltpu.sync_copy(x_hbm.at[i_vmem.at[0]], o_vmem)

        pltpu.emit_pipeline(
            body,
            grid=(m * sub // win,),
            in_specs=[pl.BlockSpec((1, win), index_map=lambda i: (0, i))],
            out_specs=[pl.BlockSpec((win, 128), index_map=lambda i: (i, 0))],
            core_axis_name=("core", "subcore"),
            dimension_semantics=(pltpu.PARALLEL,),
        )(i_hbm, o_hbm)

    return k(src2, idxe).reshape(m, c)


def _gmoe_kernel(be_ref, bv_ref, xs_ref, w1_ref, w3_ref, w2_ref, rw_ref, y_ref):
    b = pl.program_id(0)

    @pl.when(bv_ref[b] == 1)
    def _():
        xb = xs_ref[...].astype(jnp.bfloat16)
        t1 = jnp.dot(xb, w1_ref[0].astype(jnp.bfloat16),
                     preferred_element_type=jnp.float32)
        t3 = jnp.dot(xb, w3_ref[0].astype(jnp.bfloat16),
                     preferred_element_type=jnp.float32)
        g = (t1 * jax.nn.sigmoid(t1) * t3).astype(jnp.bfloat16)
        y = jnp.dot(g, w2_ref[0].astype(jnp.bfloat16),
                    preferred_element_type=jnp.float32)
        y_ref[...] = y * rw_ref[...]


def _combine_kernel(x1_ref, a_ref, b_ref, o_ref):
    o_ref[...] = (x1_ref[...] + a_ref[...].astype(jnp.float32)
                  + b_ref[...].astype(jnp.float32))


def _sel8(tbl, e):
    lane = jnp.arange(E, dtype=jnp.int32)[None, :]
    return jnp.sum(jnp.where(e[:, None] == lane, tbl[None, :], 0), axis=1)


def kernel(hidden_states, attention_mask, position_ids, ln1_w, ln2_w,
           Wq, Wk, Wv, Wo, Wr, W1, W2, W3):
    x = hidden_states.reshape(T, D)
    pos = position_ids.reshape(T, 1)
    ln1 = ln1_w.reshape(1, D)
    ln2 = ln2_w.reshape(1, D)

    nt = T // TBLK
    q2d, k2d, v2d = pl.pallas_call(
        _qkv_kernel,
        grid=(nt,),
        in_specs=[
            pl.BlockSpec((TBLK, D), lambda i: (i, 0)),
            pl.BlockSpec((TBLK, 1), lambda i: (i, 0)),
            pl.BlockSpec((1, D), lambda i: (0, 0)),
            pl.BlockSpec((D, H * HD), lambda i: (0, 0)),
            pl.BlockSpec((D, KV * HD), lambda i: (0, 0)),
            pl.BlockSpec((D, KV * HD), lambda i: (0, 0)),
        ],
        out_specs=[
            pl.BlockSpec((TBLK, H * HD), lambda i: (i, 0)),
            pl.BlockSpec((TBLK, KV * HD), lambda i: (i, 0)),
            pl.BlockSpec((TBLK, KV * HD), lambda i: (i, 0)),
        ],
        out_shape=[
            jax.ShapeDtypeStruct((T, H * HD), jnp.bfloat16),
            jax.ShapeDtypeStruct((T, KV * HD), jnp.bfloat16),
            jax.ShapeDtypeStruct((T, KV * HD), jnp.bfloat16),
        ],
    )(x, pos, ln1, Wq, Wk, Wv)

    rep = H // KV
    q3 = q2d.reshape(T, H, HD).transpose(1, 0, 2)
    k3 = k2d.reshape(T, KV, HD).transpose(1, 0, 2)
    v3 = v2d.reshape(T, KV, HD).transpose(1, 0, 2)
    attn3 = pl.pallas_call(
        _attn_kernel,
        grid=(nt, H),
        in_specs=[
            pl.BlockSpec((1, TBLK, HD), lambda i, h: (h, i, 0)),
            pl.BlockSpec((1, T, HD), lambda i, h: (h // rep, 0, 0)),
            pl.BlockSpec((1, T, HD), lambda i, h: (h // rep, 0, 0)),
        ],
        out_specs=pl.BlockSpec((1, TBLK, HD), lambda i, h: (h, i, 0)),
        out_shape=jax.ShapeDtypeStruct((H, T, HD), jnp.bfloat16),
    )(q3, k3, v3)
    attn2d = attn3.transpose(1, 0, 2).reshape(T, H * HD)

    x1, h2 = pl.pallas_call(
        _out_kernel,
        grid=(nt,),
        in_specs=[
            pl.BlockSpec((TBLK, H * HD), lambda i: (i, 0)),
            pl.BlockSpec((H * HD, D), lambda i: (0, 0)),
            pl.BlockSpec((TBLK, D), lambda i: (i, 0)),
            pl.BlockSpec((1, D), lambda i: (0, 0)),
        ],
        out_specs=[
            pl.BlockSpec((TBLK, D), lambda i: (i, 0)),
            pl.BlockSpec((TBLK, D), lambda i: (i, 0)),
        ],
        out_shape=[
            jax.ShapeDtypeStruct((T, D), jnp.float32),
            jax.ShapeDtypeStruct((T, D), jnp.float32),
        ],
    )(attn2d, Wo, x, ln2)

    ti, tw = pl.pallas_call(
        _router_kernel,
        in_specs=[
            pl.BlockSpec((T, D), lambda: (0, 0)),
            pl.BlockSpec((D, E), lambda: (0, 0)),
        ],
        out_specs=[
            pl.BlockSpec((T, TOPK), lambda: (0, 0)),
            pl.BlockSpec((T, TOPK), lambda: (0, 0)),
        ],
        out_shape=[
            jax.ShapeDtypeStruct((T, TOPK), jnp.int32),
            jax.ShapeDtypeStruct((T, TOPK), jnp.float32),
        ],
    )(h2, Wr)

    e_flat = ti.reshape(NP)
    w_flat = tw.reshape(NP)
    order = jnp.argsort(e_flat, stable=True)
    inv = jnp.zeros((NP,), jnp.int32).at[order].set(
        jnp.arange(NP, dtype=jnp.int32))
    counts = jnp.sum(
        (e_flat[:, None] == jnp.arange(E, dtype=jnp.int32)[None, :])
        .astype(jnp.int32), axis=0)
    off = jnp.concatenate([jnp.zeros((1,), jnp.int32),
                           jnp.cumsum(counts)[:-1].astype(jnp.int32)])
    cap = ((counts + RB - 1) // RB) * RB
    pad_off = jnp.concatenate([jnp.zeros((1,), jnp.int32),
                               jnp.cumsum(cap)[:-1].astype(jnp.int32)])

    j = jnp.arange(PAD_N, dtype=jnp.int32)
    e_j = jnp.sum((j[:, None] >= pad_off[None, :]).astype(jnp.int32),
                  axis=1) - 1
    cnt_j = _sel8(counts, e_j)
    within = j - _sel8(pad_off, e_j)
    valid = within < cnt_j
    src_pos = _sel8(off, e_j) + jnp.minimum(within,
                                            jnp.maximum(cnt_j - 1, 0))
    src = order[jnp.minimum(src_pos, NP - 1)]
    rows_tid = jnp.where(valid, src // TOPK, 0).astype(jnp.int32)
    rows_w = jnp.where(valid, w_flat[src], 0.0)
    blk_e = e_j[::RB]
    blk_valid = valid[::RB].astype(jnp.int32)
    dest = (_sel8(pad_off, e_flat) + inv - _sel8(off, e_flat))
    d2 = dest.reshape(T, TOPK)
    gidx = jnp.concatenate([d2[:, 0], d2[:, 1]]).astype(jnp.int32)

    xs = _sc_gather(h2, rows_tid, jnp.float32)

    y = pl.pallas_call(
        _gmoe_kernel,
        grid_spec=pltpu.PrefetchScalarGridSpec(
            num_scalar_prefetch=2,
            grid=(NBLK,),
            in_specs=[
                pl.BlockSpec((RB, D), lambda b, be, bv: (b, 0)),
                pl.BlockSpec((1, D, FF), lambda b, be, bv: (be[b], 0, 0)),
                pl.BlockSpec((1, D, FF), lambda b, be, bv: (be[b], 0, 0)),
                pl.BlockSpec((1, FF, D), lambda b, be, bv: (be[b], 0, 0)),
                pl.BlockSpec((RB, 1), lambda b, be, bv: (b, 0)),
            ],
            out_specs=pl.BlockSpec((RB, D), lambda b, be, bv: (b, 0)),
        ),
        out_shape=jax.ShapeDtypeStruct((PAD_N, D), jnp.float32),
    )(blk_e, blk_valid, xs, W1, W3, W2, rows_w.reshape(PAD_N, 1))

    yg = _sc_gather(y, gidx, jnp.float32)

    out = pl.pallas_call(
        _combine_kernel,
        grid=(nt,),
        in_specs=[
            pl.BlockSpec((TBLK, D), lambda i: (i, 0)),
            pl.BlockSpec((TBLK, D), lambda i: (i, 0)),
            pl.BlockSpec((TBLK, D), lambda i: (i + nt, 0)),
        ],
        out_specs=pl.BlockSpec((TBLK, D), lambda i: (i, 0)),
        out_shape=jax.ShapeDtypeStruct((T, D), jnp.float32),
    )(x1, yg, yg)

    return out.reshape(B, T, D)


# speedup vs baseline: 1.5274x; 1.5274x over previous
import jax
import jax.numpy as jnp
from jax.experimental import pallas as pl
from jax.experimental.pallas import tpu as pltpu

B = 1
T = 2048
D = 768
H = 12
KV = 4
HD = 64
E = 8
TOPK = 2
FF = 2048
EPS = 1e-5
THETA = 1000000.0

TBLK = 256
RB = 256
NP = T * TOPK
PAD_N = NP + E * RB
NBLK = PAD_N // RB
NEG = -1e9


def _rms(x, w):
    return x * jax.lax.rsqrt(jnp.mean(x * x, axis=-1, keepdims=True) + EPS) * w


def _qkv_kernel(x_ref, pos_ref, ln1_ref, wq_ref, wk_ref, wv_ref,
                q_ref, k_ref, v_ref):
    x = x_ref[...]
    h = _rms(x, ln1_ref[...]).astype(jnp.bfloat16)
    pos = pos_ref[...].astype(jnp.float32)
    inv_freq = 1.0 / (THETA ** (
        jax.lax.broadcasted_iota(jnp.int32, (1, HD // 2), 1).astype(jnp.float32)
        * (2.0 / HD)))
    freqs = pos * inv_freq
    emb = jnp.concatenate([freqs, freqs], axis=-1)
    cos = jnp.cos(emb)[:, None, :]
    sin = jnp.sin(emb)[:, None, :]

    def rope(y, nheads, scale):
        y3 = y.reshape(TBLK, nheads, HD)
        yr = jnp.concatenate([-y3[..., HD // 2:], y3[..., :HD // 2]], axis=-1)
        return ((y3 * cos + yr * sin) * scale).reshape(TBLK, nheads * HD)

    q = jnp.dot(h, wq_ref[...].astype(jnp.bfloat16),
                preferred_element_type=jnp.float32)
    k = jnp.dot(h, wk_ref[...].astype(jnp.bfloat16),
                preferred_element_type=jnp.float32)
    v = jnp.dot(h, wv_ref[...].astype(jnp.bfloat16),
                preferred_element_type=jnp.float32)
    q_ref[...] = rope(q, H, 1.0 / (HD ** 0.5)).astype(jnp.bfloat16)
    k_ref[...] = rope(k, KV, 1.0).astype(jnp.bfloat16)
    v_ref[...] = v.astype(jnp.bfloat16)


def _attn_kernel(q_ref, k_ref, v_ref, o_ref):
    qb = pl.program_id(0)
    q = q_ref[0]
    k = k_ref[0]
    s = jax.lax.dot_general(q, k, (((1,), (1,)), ((), ())),
                            preferred_element_type=jnp.float32)
    row = qb * TBLK + jax.lax.broadcasted_iota(jnp.int32, (TBLK, T), 0)
    col = jax.lax.broadcasted_iota(jnp.int32, (TBLK, T), 1)
    s = jnp.where(col <= row, s, NEG)
    m = jnp.max(s, axis=-1, keepdims=True)
    p = jnp.exp(s - m)
    p = p / jnp.sum(p, axis=-1, keepdims=True)
    o_ref[0] = jnp.dot(p.astype(jnp.bfloat16), v_ref[0],
                       preferred_element_type=jnp.float32).astype(jnp.bfloat16)


def _out_kernel(a_ref, wo_ref, x_ref, ln2_ref, x1_ref, h2_ref, h2b_ref):
    ao = jnp.dot(a_ref[...], wo_ref[...].astype(jnp.bfloat16),
                 preferred_element_type=jnp.float32)
    x1 = x_ref[...] + ao
    h2 = _rms(x1, ln2_ref[...])
    x1_ref[...] = x1
    h2_ref[...] = h2
    h2b_ref[...] = h2.astype(jnp.bfloat16)


def _router_kernel(h2_ref, wr_ref, ti_ref, tw_ref):
    logits = jnp.dot(h2_ref[...], wr_ref[...],
                     preferred_element_type=jnp.float32)
    m = jnp.max(logits, axis=-1, keepdims=True)
    p = jnp.exp(logits - m)
    p = p / jnp.sum(p, axis=-1, keepdims=True)
    lane = jax.lax.broadcasted_iota(jnp.int32, (T, E), 1)
    i1 = jnp.argmax(p, axis=-1, keepdims=True)
    m1 = jnp.max(p, axis=-1, keepdims=True)
    p2 = jnp.where(lane == i1, -1.0, p)
    i2 = jnp.argmax(p2, axis=-1, keepdims=True)
    m2 = jnp.max(p2, axis=-1, keepdims=True)
    denom = m1 + m2
    ti_ref[...] = jnp.concatenate([i1, i2], axis=1).astype(jnp.int32)
    tw_ref[...] = jnp.concatenate([m1 / denom, m2 / denom], axis=1)


def _gmoe_kernel(be_ref, bv_ref, d0_ref, d1_ref, h2b_ref,
                 w1_ref, w3_ref, w2_ref, y_ref):
    b = pl.program_id(0)

    @pl.when(bv_ref[b] == 1)
    def _():
        rowid = b * RB + jax.lax.broadcasted_iota(jnp.int32, (RB, T), 0)
        d0 = d0_ref[...]
        d1 = d1_ref[...]
        sel = ((d0 == rowid).astype(jnp.bfloat16)
               + (d1 == rowid).astype(jnp.bfloat16))
        xb = jnp.dot(sel, h2b_ref[...],
                     preferred_element_type=jnp.float32).astype(jnp.bfloat16)
        t1 = jnp.dot(xb, w1_ref[0].astype(jnp.bfloat16),
                     preferred_element_type=jnp.float32)
        t3 = jnp.dot(xb, w3_ref[0].astype(jnp.bfloat16),
                     preferred_element_type=jnp.float32)
        g = (t1 * jax.nn.sigmoid(t1) * t3).astype(jnp.bfloat16)
        y = jnp.dot(g, w2_ref[0].astype(jnp.bfloat16),
                    preferred_element_type=jnp.float32)
        y_ref[...] = y.astype(jnp.bfloat16)

    @pl.when(bv_ref[b] == 0)
    def _():
        y_ref[...] = jnp.zeros((RB, D), jnp.bfloat16)


def _combine_kernel(x1_ref, d0_ref, d1_ref, w0_ref, w1_ref, y_ref, o_ref):
    colid = jax.lax.broadcasted_iota(jnp.int32, (TBLK, PAD_N), 1)
    d0 = d0_ref[...]
    d1 = d1_ref[...]
    w0 = w0_ref[...]
    w1 = w1_ref[...]
    s2 = (jnp.where(d0 == colid, w0, 0.0)
          + jnp.where(d1 == colid, w1, 0.0)).astype(jnp.bfloat16)
    moe = jnp.dot(s2, y_ref[...], preferred_element_type=jnp.float32)
    o_ref[...] = x1_ref[...] + moe


def _sel8(tbl, e):
    lane = jnp.arange(E, dtype=jnp.int32)[None, :]
    return jnp.sum(jnp.where(e[:, None] == lane, tbl[None, :], 0), axis=1)


def kernel(hidden_states, attention_mask, position_ids, ln1_w, ln2_w,
           Wq, Wk, Wv, Wo, Wr, W1, W2, W3):
    x = hidden_states.reshape(T, D)
    pos = position_ids.reshape(T, 1)
    ln1 = ln1_w.reshape(1, D)
    ln2 = ln2_w.reshape(1, D)

    nt = T // TBLK
    q2d, k2d, v2d = pl.pallas_call(
        _qkv_kernel,
        grid=(nt,),
        in_specs=[
            pl.BlockSpec((TBLK, D), lambda i: (i, 0)),
            pl.BlockSpec((TBLK, 1), lambda i: (i, 0)),
            pl.BlockSpec((1, D), lambda i: (0, 0)),
            pl.BlockSpec((D, H * HD), lambda i: (0, 0)),
            pl.BlockSpec((D, KV * HD), lambda i: (0, 0)),
            pl.BlockSpec((D, KV * HD), lambda i: (0, 0)),
        ],
        out_specs=[
            pl.BlockSpec((TBLK, H * HD), lambda i: (i, 0)),
            pl.BlockSpec((TBLK, KV * HD), lambda i: (i, 0)),
            pl.BlockSpec((TBLK, KV * HD), lambda i: (i, 0)),
        ],
        out_shape=[
            jax.ShapeDtypeStruct((T, H * HD), jnp.bfloat16),
            jax.ShapeDtypeStruct((T, KV * HD), jnp.bfloat16),
            jax.ShapeDtypeStruct((T, KV * HD), jnp.bfloat16),
        ],
    )(x, pos, ln1, Wq, Wk, Wv)

    rep = H // KV
    q3 = q2d.reshape(T, H, HD).transpose(1, 0, 2)
    k3 = k2d.reshape(T, KV, HD).transpose(1, 0, 2)
    v3 = v2d.reshape(T, KV, HD).transpose(1, 0, 2)
    attn3 = pl.pallas_call(
        _attn_kernel,
        grid=(nt, H),
        in_specs=[
            pl.BlockSpec((1, TBLK, HD), lambda i, h: (h, i, 0)),
            pl.BlockSpec((1, T, HD), lambda i, h: (h // rep, 0, 0)),
            pl.BlockSpec((1, T, HD), lambda i, h: (h // rep, 0, 0)),
        ],
        out_specs=pl.BlockSpec((1, TBLK, HD), lambda i, h: (h, i, 0)),
        out_shape=jax.ShapeDtypeStruct((H, T, HD), jnp.bfloat16),
    )(q3, k3, v3)
    attn2d = attn3.transpose(1, 0, 2).reshape(T, H * HD)

    x1, h2, h2b = pl.pallas_call(
        _out_kernel,
        grid=(nt,),
        in_specs=[
            pl.BlockSpec((TBLK, H * HD), lambda i: (i, 0)),
            pl.BlockSpec((H * HD, D), lambda i: (0, 0)),
            pl.BlockSpec((TBLK, D), lambda i: (i, 0)),
            pl.BlockSpec((1, D), lambda i: (0, 0)),
        ],
        out_specs=[
            pl.BlockSpec((TBLK, D), lambda i: (i, 0)),
            pl.BlockSpec((TBLK, D), lambda i: (i, 0)),
            pl.BlockSpec((TBLK, D), lambda i: (i, 0)),
        ],
        out_shape=[
            jax.ShapeDtypeStruct((T, D), jnp.float32),
            jax.ShapeDtypeStruct((T, D), jnp.float32),
            jax.ShapeDtypeStruct((T, D), jnp.bfloat16),
        ],
    )(attn2d, Wo, x, ln2)

    ti, tw = pl.pallas_call(
        _router_kernel,
        in_specs=[
            pl.BlockSpec((T, D), lambda: (0, 0)),
            pl.BlockSpec((D, E), lambda: (0, 0)),
        ],
        out_specs=[
            pl.BlockSpec((T, TOPK), lambda: (0, 0)),
            pl.BlockSpec((T, TOPK), lambda: (0, 0)),
        ],
        out_shape=[
            jax.ShapeDtypeStruct((T, TOPK), jnp.int32),
            jax.ShapeDtypeStruct((T, TOPK), jnp.float32),
        ],
    )(h2, Wr)

    e_flat = ti.reshape(NP)
    lane8 = jnp.arange(E, dtype=jnp.int32)[None, :]
    oh = (e_flat[:, None] == lane8).astype(jnp.int32)
    csum = jnp.cumsum(oh, axis=0)
    rank = jnp.sum(oh * csum, axis=1) - 1
    counts = csum[-1]
    cap = ((counts + RB - 1) // RB) * RB
    pad_off = jnp.concatenate([jnp.zeros((1,), jnp.int32),
                               jnp.cumsum(cap)[:-1].astype(jnp.int32)])
    dest = _sel8(pad_off, e_flat) + rank
    d2 = dest.reshape(T, TOPK)
    dest0 = d2[:, 0:1].astype(jnp.int32)
    dest1 = d2[:, 1:2].astype(jnp.int32)
    jb = jnp.arange(NBLK, dtype=jnp.int32) * RB
    blk_e = (jnp.sum((jb[:, None] >= pad_off[None, :]).astype(jnp.int32),
                     axis=1) - 1).astype(jnp.int32)
    blk_valid = ((jb - _sel8(pad_off, blk_e))
                 < _sel8(counts, blk_e)).astype(jnp.int32)

    y = pl.pallas_call(
        _gmoe_kernel,
        grid_spec=pltpu.PrefetchScalarGridSpec(
            num_scalar_prefetch=2,
            grid=(NBLK,),
            in_specs=[
                pl.BlockSpec((1, T), lambda b, be, bv: (0, 0)),
                pl.BlockSpec((1, T), lambda b, be, bv: (0, 0)),
                pl.BlockSpec((T, D), lambda b, be, bv: (0, 0)),
                pl.BlockSpec((1, D, FF), lambda b, be, bv: (be[b], 0, 0)),
                pl.BlockSpec((1, D, FF), lambda b, be, bv: (be[b], 0, 0)),
                pl.BlockSpec((1, FF, D), lambda b, be, bv: (be[b], 0, 0)),
            ],
            out_specs=pl.BlockSpec((RB, D), lambda b, be, bv: (b, 0)),
        ),
        out_shape=jax.ShapeDtypeStruct((PAD_N, D), jnp.bfloat16),
    )(blk_e, blk_valid, dest0.reshape(1, T), dest1.reshape(1, T),
      h2b, W1, W3, W2)

    out = pl.pallas_call(
        _combine_kernel,
        grid=(nt,),
        in_specs=[
            pl.BlockSpec((TBLK, D), lambda i: (i, 0)),
            pl.BlockSpec((TBLK, 1), lambda i: (i, 0)),
            pl.BlockSpec((TBLK, 1), lambda i: (i, 0)),
            pl.BlockSpec((TBLK, 1), lambda i: (i, 0)),
            pl.BlockSpec((TBLK, 1), lambda i: (i, 0)),
            pl.BlockSpec((PAD_N, D), lambda i: (0, 0)),
        ],
        out_specs=pl.BlockSpec((TBLK, D), lambda i: (i, 0)),
        out_shape=jax.ShapeDtypeStruct((T, D), jnp.float32),
    )(x1, dest0, dest1, tw[:, 0:1], tw[:, 1:2], y)

    return out.reshape(B, T, D)
